# baseline (speedup 1.0000x reference)
import jax
import jax.numpy as jnp
from jax.experimental import pallas as pl
from jax.experimental.pallas import tpu as pltpu


def _s1_kernel(x_ref, w1_ref, o_ref):
    o_ref[...] = jnp.dot(x_ref[...], w1_ref[...],
                         preferred_element_type=jnp.float32)


def _layer1_kernel(adj_ref, s1_ref, b1_ref, w2_ref, o_ref):
    acc = jnp.dot(adj_ref[...].astype(jnp.bfloat16),
                  s1_ref[...].astype(jnp.bfloat16),
                  preferred_element_type=jnp.float32)
    h = jnp.maximum(acc + b1_ref[...], 0.0)
    o_ref[...] = jnp.dot(h, w2_ref[...], preferred_element_type=jnp.float32)


def _layer2_kernel(adj_ref, t2_ref, b2_ref, o_ref):
    z = jnp.dot(adj_ref[...].astype(jnp.bfloat16),
                t2_ref[...].astype(jnp.bfloat16),
                preferred_element_type=jnp.float32) + b2_ref[...]
    m = jnp.max(z, axis=1, keepdims=True)
    e = jnp.exp(z - m)
    lse = jnp.log(jnp.sum(e, axis=1, keepdims=True)) + m
    o_ref[...] = z - lse


def kernel(x, adj, W1, b1, W2, b2):
    n, nfeat = x.shape
    nhid = W1.shape[1]
    ncls = W2.shape[1]
    BM = 200
    BS1 = 2000
    b1r = b1.reshape(1, nhid)
    b2r = b2.reshape(1, ncls)

    s1 = pl.pallas_call(
        _s1_kernel,
        grid=(n // BS1,),
        in_specs=[pl.BlockSpec((BS1, nfeat), lambda i: (i, 0)),
                  pl.BlockSpec((nfeat, nhid), lambda i: (0, 0))],
        out_specs=pl.BlockSpec((BS1, nhid), lambda i: (i, 0)),
        out_shape=jax.ShapeDtypeStruct((n, nhid), jnp.float32),
    )(x, W1)

    t2 = pl.pallas_call(
        _layer1_kernel,
        grid=(n // BM,),
        in_specs=[pl.BlockSpec((BM, n), lambda i: (i, 0)),
                  pl.BlockSpec((n, nhid), lambda i: (0, 0)),
                  pl.BlockSpec((1, nhid), lambda i: (0, 0)),
                  pl.BlockSpec((nhid, ncls), lambda i: (0, 0))],
        out_specs=pl.BlockSpec((BM, ncls), lambda i: (i, 0)),
        out_shape=jax.ShapeDtypeStruct((n, ncls), jnp.float32),
        compiler_params=pltpu.CompilerParams(
            dimension_semantics=("arbitrary",)),
    )(adj, s1, b1r, W2)

    out = pl.pallas_call(
        _layer2_kernel,
        grid=(n // BM,),
        in_specs=[pl.BlockSpec((BM, n), lambda i: (i, 0)),
                  pl.BlockSpec((n, ncls), lambda i: (0, 0)),
                  pl.BlockSpec((1, ncls), lambda i: (0, 0))],
        out_specs=pl.BlockSpec((BM, ncls), lambda i: (i, 0)),
        out_shape=jax.ShapeDtypeStruct((n, ncls), jnp.float32),
        compiler_params=pltpu.CompilerParams(
            dimension_semantics=("arbitrary",)),
    )(adj, t2, b2r)
    return out


# speedup vs baseline: 1.0278x; 1.0278x over previous
import jax
import jax.numpy as jnp
from jax.experimental import pallas as pl
from jax.experimental.pallas import tpu as pltpu


def _s1_kernel(x_ref, w1_ref, o_ref):
    o_ref[...] = jnp.dot(x_ref[...], w1_ref[...],
                         preferred_element_type=jnp.float32)


def _layer1_kernel(adj_ref, s1_ref, b1_ref, w2_ref, o_ref):
    acc = jnp.dot(adj_ref[...].astype(jnp.bfloat16),
                  s1_ref[...].astype(jnp.bfloat16),
                  preferred_element_type=jnp.float32)
    h = jnp.maximum(acc + b1_ref[...], 0.0)
    o_ref[...] = jnp.dot(h, w2_ref[...], preferred_element_type=jnp.float32)


def _layer2_kernel(adj_ref, t2_ref, b2_ref, o_ref):
    z = jnp.dot(adj_ref[...].astype(jnp.bfloat16),
                t2_ref[...].astype(jnp.bfloat16),
                preferred_element_type=jnp.float32) + b2_ref[...]
    m = jnp.max(z, axis=1, keepdims=True)
    e = jnp.exp(z - m)
    lse = jnp.log(jnp.sum(e, axis=1, keepdims=True)) + m
    o_ref[...] = z - lse


def kernel(x, adj, W1, b1, W2, b2):
    n, nfeat = x.shape
    nhid = W1.shape[1]
    ncls = W2.shape[1]
    BM = 400
    BS1 = 2000
    b1r = b1.reshape(1, nhid)
    b2r = b2.reshape(1, ncls)

    s1 = pl.pallas_call(
        _s1_kernel,
        grid=(n // BS1,),
        in_specs=[pl.BlockSpec((BS1, nfeat), lambda i: (i, 0)),
                  pl.BlockSpec((nfeat, nhid), lambda i: (0, 0))],
        out_specs=pl.BlockSpec((BS1, nhid), lambda i: (i, 0)),
        out_shape=jax.ShapeDtypeStruct((n, nhid), jnp.float32),
    )(x, W1)

    t2 = pl.pallas_call(
        _layer1_kernel,
        grid=(n // BM,),
        in_specs=[pl.BlockSpec((BM, n), lambda i: (i, 0)),
                  pl.BlockSpec((n, nhid), lambda i: (0, 0)),
                  pl.BlockSpec((1, nhid), lambda i: (0, 0)),
                  pl.BlockSpec((nhid, ncls), lambda i: (0, 0))],
        out_specs=pl.BlockSpec((BM, ncls), lambda i: (i, 0)),
        out_shape=jax.ShapeDtypeStruct((n, ncls), jnp.float32),
        compiler_params=pltpu.CompilerParams(
            dimension_semantics=("arbitrary",)),
    )(adj, s1, b1r, W2)

    out = pl.pallas_call(
        _layer2_kernel,
        grid=(n // BM,),
        in_specs=[pl.BlockSpec((BM, n), lambda i: (i, 0)),
                  pl.BlockSpec((n, ncls), lambda i: (0, 0)),
                  pl.BlockSpec((1, ncls), lambda i: (0, 0))],
        out_specs=pl.BlockSpec((BM, ncls), lambda i: (i, 0)),
        out_shape=jax.ShapeDtypeStruct((n, ncls), jnp.float32),
        compiler_params=pltpu.CompilerParams(
            dimension_semantics=("arbitrary",)),
    )(adj, t2, b2r)
    return out


# speedup vs baseline: 1.0745x; 1.0454x over previous
import jax
import jax.numpy as jnp
from jax.experimental import pallas as pl
from jax.experimental.pallas import tpu as pltpu


def _fused_kernel(adj_ref, x_ref, w1_ref, b1_ref, w2_ref, b2_ref,
                  o_ref, s1_ref, t2_ref):
    p = pl.program_id(0)
    i = pl.program_id(1)
    bm = o_ref.shape[0]

    @pl.when((p == 0) & (i == 0))
    def _compute_s1():
        s1_ref[...] = jnp.dot(x_ref[...], w1_ref[...],
                              preferred_element_type=jnp.float32
                              ).astype(jnp.bfloat16)

    @pl.when(p == 0)
    def _layer1():
        acc = jnp.dot(adj_ref[...].astype(jnp.bfloat16), s1_ref[...],
                      preferred_element_type=jnp.float32)
        h = jnp.maximum(acc + b1_ref[...], 0.0)
        t2_ref[pl.ds(i * bm, bm), :] = jnp.dot(
            h, w2_ref[...], preferred_element_type=jnp.float32
        ).astype(jnp.bfloat16)

    @pl.when(p == 1)
    def _layer2():
        z = jnp.dot(adj_ref[...].astype(jnp.bfloat16), t2_ref[...],
                    preferred_element_type=jnp.float32) + b2_ref[...]
        m = jnp.max(z, axis=1, keepdims=True)
        e = jnp.exp(z - m)
        lse = jnp.log(jnp.sum(e, axis=1, keepdims=True)) + m
        o_ref[...] = z - lse


def kernel(x, adj, W1, b1, W2, b2):
    n, nfeat = x.shape
    nhid = W1.shape[1]
    ncls = W2.shape[1]
    BM = 400
    b1r = b1.reshape(1, nhid)
    b2r = b2.reshape(1, ncls)

    return pl.pallas_call(
        _fused_kernel,
        grid=(2, n // BM),
        in_specs=[pl.BlockSpec((BM, n), lambda p, i: (i, 0)),
                  pl.BlockSpec((n, nfeat), lambda p, i: (0, 0)),
                  pl.BlockSpec((nfeat, nhid), lambda p, i: (0, 0)),
                  pl.BlockSpec((1, nhid), lambda p, i: (0, 0)),
                  pl.BlockSpec((nhid, ncls), lambda p, i: (0, 0)),
                  pl.BlockSpec((1, ncls), lambda p, i: (0, 0))],
        out_specs=pl.BlockSpec((BM, ncls), lambda p, i: (i, 0)),
        out_shape=jax.ShapeDtypeStruct((n, ncls), jnp.float32),
        scratch_shapes=[pltpu.VMEM((n, nhid), jnp.bfloat16),
                        pltpu.VMEM((n, ncls), jnp.bfloat16)],
        compiler_params=pltpu.CompilerParams(
            dimension_semantics=("arbitrary", "arbitrary")),
    )(adj, x, W1, b1r, W2, b2r)
